# baseline (speedup 1.0000x reference)
import functools

import jax
import jax.numpy as jnp
from jax import lax
from jax.experimental import pallas as pl
from jax.experimental.pallas import tpu as pltpu
from jax.experimental.pallas import tpu_sc as plsc

_NC = 2
_NS = 16
_LANES = 16


def _degree_sc(edge_index, n_nodes, n_edges):
    nw = _NC * _NS
    unit = 128
    main = (n_edges // (nw * unit)) * unit
    rem_units = (n_edges - nw * main) // unit
    f32t = 128
    n_pad = ((n_nodes + _NS * f32t - 1) // (_NS * f32t)) * (_NS * f32t)
    z_chunk = n_pad // _NS
    mesh = plsc.VectorSubcoreMesh(
        core_axis_name="c", subcore_axis_name="s",
        num_cores=_NC, num_subcores=_NS)

    @functools.partial(
        pl.kernel,
        out_type=jax.ShapeDtypeStruct((_NC * n_pad,), jnp.float32),
        mesh=mesh,
        scratch_types=[
            pltpu.VMEM((2, main), jnp.int32),
            pltpu.VMEM((2, unit), jnp.int32),
            pltpu.VMEM((unit,), jnp.float32),
            pltpu.VMEM((n_pad,), jnp.float32),
            pltpu.VMEM_SHARED((n_pad,), jnp.float32),
            pltpu.SemaphoreType.DMA,
            pltpu.SemaphoreType.DMA,
        ],
    )
    def deg_kernel(ei_hbm, ones_hbm, zeros_hbm, out_hbm,
                   idx_v, ext_v, ones_v, buf_v, deg_sh, sem, sem2):
        c = lax.axis_index("c")
        s = lax.axis_index("s")
        wid = c * _NS + s
        n_units = main // unit

        pltpu.async_copy(ei_hbm.at[:, pl.ds(wid * main, main)], idx_v, sem2)

        @pl.when(wid < rem_units)
        def _stage_rem():
            pltpu.async_copy(
                ei_hbm.at[:, pl.ds(nw * main + wid * unit, unit)], ext_v,
                sem2)

        pltpu.sync_copy(ones_hbm, ones_v)
        pltpu.sync_copy(zeros_hbm, deg_sh.at[pl.ds(s * z_chunk, z_chunk)])

        pltpu.make_async_copy(
            ei_hbm.at[:, pl.ds(wid * main, main)], idx_v, sem2).wait()

        @pl.when(wid < rem_units)
        def _stage_rem_wait():
            pltpu.make_async_copy(
                ei_hbm.at[:, pl.ds(nw * main + wid * unit, unit)], ext_v,
                sem2).wait()

        plsc.subcore_barrier()
        def fire(u, carry):
            pltpu.async_copy(
                ones_v, deg_sh.at[idx_v.at[0, pl.ds(u * unit, unit)]],
                sem, add=True)
            return carry
        lax.fori_loop(0, n_units, fire, 0)

        @pl.when(wid < rem_units)
        def _scatter_rem():
            pltpu.async_copy(ones_v, deg_sh.at[ext_v.at[0]], sem, add=True)

        def drain(u, carry):
            pltpu.make_async_copy(
                ones_v, deg_sh.at[idx_v.at[0, pl.ds(0, unit)]], sem).wait()
            return carry
        lax.fori_loop(0, n_units, drain, 0)

        @pl.when(wid < rem_units)
        def _drain_rem():
            pltpu.make_async_copy(
                ones_v, deg_sh.at[ext_v.at[0]], sem).wait()

        plsc.subcore_barrier()

        @pl.when(s == 0)
        def _writeout():
            pltpu.sync_copy(deg_sh, buf_v)
            pltpu.sync_copy(buf_v, out_hbm.at[pl.ds(c * n_pad, n_pad)])

    return deg_kernel(edge_index,
                      jnp.ones((unit,), jnp.float32),
                      jnp.zeros((z_chunk,), jnp.float32))


def _scale_body(x_ref, d_ref, t1_ref, t2_ref, o_ref):
    deg = d_ref[0, 0, 0] + d_ref[1, 0, 0]
    scale = jnp.log(1.0 + deg).reshape(-1, 1)
    o_ref[...] = x_ref[...] * (t1_ref[...] + scale * t2_ref[...])


def kernel(x, edge_index, theta1, theta2):
    n_nodes, hidden = x.shape
    n_edges = edge_index.shape[1]
    ei = edge_index.astype(jnp.int32)

    deg_flat = _degree_sc(ei, n_nodes, n_edges)
    grid = 1
    blk = n_nodes // grid
    n_pad = deg_flat.shape[0] // _NC
    d4 = deg_flat.reshape(_NC, n_pad)[:, :n_nodes].reshape(
        _NC, grid, 1, blk)

    out = pl.pallas_call(
        _scale_body,
        grid=(grid,),
        in_specs=[
            pl.BlockSpec((blk, hidden), lambda i: (i, 0)),
            pl.BlockSpec((_NC, 1, 1, blk), lambda i: (0, i, 0, 0)),
            pl.BlockSpec((1, hidden), lambda i: (0, 0)),
            pl.BlockSpec((1, hidden), lambda i: (0, 0)),
        ],
        out_specs=pl.BlockSpec((blk, hidden), lambda i: (i, 0)),
        out_shape=jax.ShapeDtypeStruct((n_nodes, hidden), jnp.float32),
    )(x, d4, theta1.reshape(1, hidden), theta2.reshape(1, hidden))
    return out


# speedup vs baseline: 1.0744x; 1.0744x over previous
import functools

import jax
import jax.numpy as jnp
from jax import lax
from jax.experimental import pallas as pl
from jax.experimental.pallas import tpu as pltpu
from jax.experimental.pallas import tpu_sc as plsc

_NC = 2
_NS = 16
_LANES = 16


def _degree_sc(edge_index, n_nodes, n_edges):
    nw = _NC * _NS
    unit = 128
    main = (n_edges // (nw * unit)) * unit
    rem_units = (n_edges - nw * main) // unit
    f32t = 128
    n_pad = ((n_nodes + _NS * f32t - 1) // (_NS * f32t)) * (_NS * f32t)
    z_chunk = n_pad // _NS
    mesh = plsc.VectorSubcoreMesh(
        core_axis_name="c", subcore_axis_name="s",
        num_cores=_NC, num_subcores=_NS)

    @functools.partial(
        pl.kernel,
        out_type=jax.ShapeDtypeStruct((_NC * n_pad,), jnp.float32),
        mesh=mesh,
        scratch_types=[
            pltpu.VMEM((2, main), jnp.int32),
            pltpu.VMEM((2, unit), jnp.int32),
            pltpu.VMEM((unit,), jnp.float32),
            pltpu.VMEM((n_pad,), jnp.float32),
            pltpu.VMEM_SHARED((n_pad,), jnp.float32),
            pltpu.SemaphoreType.DMA,
            pltpu.SemaphoreType.DMA,
        ],
    )
    def deg_kernel(ei_hbm, out_hbm,
                   idx_v, ext_v, ones_v, buf_v, deg_sh, sem, sem2):
        c = lax.axis_index("c")
        s = lax.axis_index("s")
        wid = c * _NS + s
        n_units = main // unit

        pltpu.async_copy(ei_hbm.at[:, pl.ds(wid * main, main)], idx_v, sem2)

        @pl.when(wid < rem_units)
        def _stage_rem():
            pltpu.async_copy(
                ei_hbm.at[:, pl.ds(nw * main + wid * unit, unit)], ext_v,
                sem2)

        for k in range(unit // _LANES):
            ones_v[pl.ds(k * _LANES, _LANES)] = jnp.ones(
                (_LANES,), jnp.float32)
        for k in range(z_chunk // _LANES):
            buf_v[pl.ds(k * _LANES, _LANES)] = jnp.zeros(
                (_LANES,), jnp.float32)
        pltpu.sync_copy(buf_v.at[pl.ds(0, z_chunk)],
                        deg_sh.at[pl.ds(s * z_chunk, z_chunk)])

        pltpu.make_async_copy(
            ei_hbm.at[:, pl.ds(wid * main, main)], idx_v, sem2).wait()

        @pl.when(wid < rem_units)
        def _stage_rem_wait():
            pltpu.make_async_copy(
                ei_hbm.at[:, pl.ds(nw * main + wid * unit, unit)], ext_v,
                sem2).wait()

        plsc.subcore_barrier()
        def fire(u, carry):
            pltpu.async_copy(
                ones_v, deg_sh.at[idx_v.at[0, pl.ds(u * unit, unit)]],
                sem, add=True)
            return carry
        lax.fori_loop(0, n_units, fire, 0)

        @pl.when(wid < rem_units)
        def _scatter_rem():
            pltpu.async_copy(ones_v, deg_sh.at[ext_v.at[0]], sem, add=True)

        def drain(u, carry):
            pltpu.make_async_copy(
                ones_v, deg_sh.at[idx_v.at[0, pl.ds(0, unit)]], sem).wait()
            return carry
        lax.fori_loop(0, n_units, drain, 0)

        @pl.when(wid < rem_units)
        def _drain_rem():
            pltpu.make_async_copy(
                ones_v, deg_sh.at[ext_v.at[0]], sem).wait()

        plsc.subcore_barrier()

        @pl.when(s == 0)
        def _writeout():
            pltpu.sync_copy(deg_sh, buf_v)
            pltpu.sync_copy(buf_v, out_hbm.at[pl.ds(c * n_pad, n_pad)])

    return deg_kernel(edge_index)


def _scale_body(x_ref, d_ref, t1_ref, t2_ref, o_ref):
    deg = d_ref[0, 0, 0] + d_ref[1, 0, 0]
    scale = jnp.log(1.0 + deg).reshape(-1, 1)
    o_ref[...] = x_ref[...] * (t1_ref[...] + scale * t2_ref[...])


def kernel(x, edge_index, theta1, theta2):
    n_nodes, hidden = x.shape
    n_edges = edge_index.shape[1]
    ei = edge_index.astype(jnp.int32)

    deg_flat = _degree_sc(ei, n_nodes, n_edges)
    grid = 2
    blk = n_nodes // grid
    n_pad = deg_flat.shape[0] // _NC
    d4 = deg_flat.reshape(_NC, n_pad)[:, :n_nodes].reshape(
        _NC, grid, 1, blk)

    out = pl.pallas_call(
        _scale_body,
        grid=(grid,),
        in_specs=[
            pl.BlockSpec((blk, hidden), lambda i: (i, 0)),
            pl.BlockSpec((_NC, 1, 1, blk), lambda i: (0, i, 0, 0)),
            pl.BlockSpec((1, hidden), lambda i: (0, 0)),
            pl.BlockSpec((1, hidden), lambda i: (0, 0)),
        ],
        out_specs=pl.BlockSpec((blk, hidden), lambda i: (i, 0)),
        out_shape=jax.ShapeDtypeStruct((n_nodes, hidden), jnp.float32),
    )(x, d4, theta1.reshape(1, hidden), theta2.reshape(1, hidden))
    return out
